# Initial kernel scaffold; baseline (speedup 1.0000x reference)
import jax
import jax.numpy as jnp
from jax.experimental import pallas as pl


def kernel(input, embed):
    raise NotImplementedError("write your pallas kernel here")


# speedup vs baseline: 1.3461x; 1.3461x over previous
import functools

import jax
import jax.numpy as jnp
from jax import lax
from jax.experimental import pallas as pl
from jax.experimental.pallas import tpu as pltpu

try:
    from jax.experimental.pallas import tpu_sc as plsc
except ImportError:
    plsc = None

_TN = 512
_TK = 512


def _codes_body(x_ref, e_ref, out_ref, *, n_rows, k_total):
    x = x_ref[...]
    x2 = jnp.sum(x * x, axis=1, keepdims=True)
    m = jnp.full((n_rows, 1), jnp.inf, dtype=jnp.float32)
    a = jnp.zeros((n_rows, 1), dtype=jnp.int32)
    for j in range(k_total // _TK):
        e = e_ref[j * _TK:(j + 1) * _TK, :]
        e2 = jnp.sum(e * e, axis=1)[None, :]
        xe2 = lax.dot_general(x, e * (-2.0),
                              (((1,), (1,)), ((), ())),
                              preferred_element_type=jnp.float32)
        s = (x2 + e2) + xe2
        lmin = jnp.min(s, axis=1, keepdims=True)
        ii = lax.broadcasted_iota(jnp.int32, (n_rows, _TK), 1)
        larg = jnp.min(jnp.where(s == lmin, ii, _TK), axis=1, keepdims=True)
        larg = larg + j * _TK
        take = lmin < m
        m = jnp.where(take, lmin, m)
        a = jnp.where(take, larg, a)
    out_ref[...] = a


def _codes_tc(x, embed, *, interpret=False):
    n, d = x.shape
    k, _ = embed.shape
    grid = (n // _TN,)
    return pl.pallas_call(
        functools.partial(_codes_body, n_rows=_TN, k_total=k),
        grid=grid,
        in_specs=[
            pl.BlockSpec((_TN, d), lambda i: (i, 0)),
            pl.BlockSpec((k, d), lambda i: (0, 0)),
        ],
        out_specs=pl.BlockSpec((_TN, 1), lambda i: (i, 0)),
        out_shape=jax.ShapeDtypeStruct((n, 1), jnp.int32),
        interpret=interpret,
    )(x, embed)


def _gather_sc(embed, codes):
    k, d = embed.shape
    n = codes.shape[0]
    info = plsc.get_sparse_core_info()
    nc, ns = info.num_cores, info.num_subcores
    nw = nc * ns
    b_per_w = n // nw
    ch = 96
    nch = b_per_w // ch
    mesh = plsc.VectorSubcoreMesh(core_axis_name="c", subcore_axis_name="s")

    @functools.partial(
        pl.kernel,
        mesh=mesh,
        out_type=jax.ShapeDtypeStruct((n, d), jnp.float32),
        scratch_types=[
            pltpu.VMEM((b_per_w,), jnp.int32),
            pltpu.VMEM((b_per_w, d), jnp.float32),
            pltpu.SemaphoreType.DMA,
        ],
    )
    def gather_kernel(table_hbm, idx_hbm, out_hbm, idx_v, rows_v, sem):
        wid = lax.axis_index("s") * nc + lax.axis_index("c")
        base = wid * b_per_w
        pltpu.sync_copy(idx_hbm.at[pl.ds(base, b_per_w)], idx_v)
        cps = []
        for c in range(nch):
            cps.append(pltpu.async_copy(
                table_hbm.at[idx_v.at[pl.ds(c * ch, ch)]],
                rows_v.at[pl.ds(c * ch, ch)], sem))
        for cp in cps:
            cp.wait()
        pltpu.sync_copy(rows_v, out_hbm.at[pl.ds(base, b_per_w)])

    return gather_kernel(embed, codes)


def kernel(input, embed):
    b, n, d = input.shape
    x = input.reshape(b * n, d)
    codes = _codes_tc(x, embed).reshape(b * n)
    quantize = _gather_sc(embed, codes)
    return quantize.reshape(b, n, d), codes.reshape(b, n)


# speedup vs baseline: 1.5242x; 1.1324x over previous
import functools

import jax
import jax.numpy as jnp
from jax import lax
from jax.experimental import pallas as pl
from jax.experimental.pallas import tpu as pltpu

try:
    from jax.experimental.pallas import tpu_sc as plsc
except ImportError:
    plsc = None

_TN = 512
_TK = 512


def _codes_body(x_ref, e_ref, out_ref, *, n_rows, k_total):
    x = x_ref[...]
    x2 = jnp.sum(x * x, axis=1, keepdims=True)
    m = jnp.full((n_rows, 1), jnp.inf, dtype=jnp.float32)
    a = jnp.zeros((n_rows, 1), dtype=jnp.float32)
    ii = lax.broadcasted_iota(jnp.int32, (n_rows, _TK), 1).astype(jnp.float32)
    for j in range(k_total // _TK):
        e = e_ref[j * _TK:(j + 1) * _TK, :]
        e2 = jnp.sum(e * e, axis=1)[None, :]
        xe2 = lax.dot_general(x, e * (-2.0),
                              (((1,), (1,)), ((), ())),
                              preferred_element_type=jnp.float32)
        s = (x2 + e2) + xe2
        lmin = jnp.min(s, axis=1, keepdims=True)
        larg = jnp.min(jnp.where(s == lmin, ii, float(_TK)),
                       axis=1, keepdims=True)
        larg = larg + float(j * _TK)
        take = lmin < m
        m = jnp.where(take, lmin, m)
        a = jnp.where(take, larg, a)
    out_ref[...] = a.astype(jnp.int32)


def _codes_tc(x, embed, *, interpret=False):
    n, d = x.shape
    k, _ = embed.shape
    grid = (n // _TN,)
    return pl.pallas_call(
        functools.partial(_codes_body, n_rows=_TN, k_total=k),
        grid=grid,
        in_specs=[
            pl.BlockSpec((_TN, d), lambda i: (i, 0)),
            pl.BlockSpec((k, d), lambda i: (0, 0)),
        ],
        out_specs=pl.BlockSpec((_TN, 1), lambda i: (i, 0)),
        out_shape=jax.ShapeDtypeStruct((n, 1), jnp.int32),
        interpret=interpret,
    )(x, embed)


def _gather_sc(embed, codes):
    k, d = embed.shape
    n = codes.shape[0]
    info = plsc.get_sparse_core_info()
    nc, ns = info.num_cores, info.num_subcores
    nw = nc * ns
    b_per_w = n // nw
    ch = 96
    nch = b_per_w // ch
    mesh = plsc.VectorSubcoreMesh(core_axis_name="c", subcore_axis_name="s")

    @functools.partial(
        pl.kernel,
        mesh=mesh,
        out_type=jax.ShapeDtypeStruct((n, d), jnp.float32),
        scratch_types=[
            pltpu.VMEM((b_per_w,), jnp.int32),
            pltpu.VMEM((b_per_w, d), jnp.float32),
            pltpu.SemaphoreType.DMA,
        ],
    )
    def gather_kernel(table_hbm, idx_hbm, out_hbm, idx_v, rows_v, sem):
        wid = lax.axis_index("s") * nc + lax.axis_index("c")
        base = wid * b_per_w
        pltpu.sync_copy(idx_hbm.at[pl.ds(base, b_per_w)], idx_v)
        cps = []
        for c in range(nch):
            cps.append(pltpu.async_copy(
                table_hbm.at[idx_v.at[pl.ds(c * ch, ch)]],
                rows_v.at[pl.ds(c * ch, ch)], sem))
        for cp in cps:
            cp.wait()
        pltpu.sync_copy(rows_v, out_hbm.at[pl.ds(base, b_per_w)])

    return gather_kernel(embed, codes)


def kernel(input, embed):
    b, n, d = input.shape
    x = input.reshape(b * n, d)
    codes = _codes_tc(x, embed).reshape(b * n)
    quantize = _gather_sc(embed, codes)
    return quantize.reshape(b, n, d), codes.reshape(b, n)
